# baseline (speedup 1.0000x reference)
import functools

import jax
import jax.numpy as jnp
from jax import lax
from jax.experimental import pallas as pl
from jax.experimental.pallas import tpu as pltpu
from jax.experimental.pallas import tpu_sc as plsc

NC = 2
NS = 16
LANES = 16
IDX_CHUNK = 128
W = 48
CH = 800

_SC_PARAMS = pltpu.CompilerParams(
    needs_layout_passes=False, use_tc_tiling_on_sc=False)


def _build_fuser(NI, D):
    NW = NC * NS
    NCH = NI // CH
    ITER = -(-NCH // NW)
    mesh = plsc.VectorSubcoreMesh(core_axis_name="c", subcore_axis_name="s")

    @functools.partial(
        pl.kernel,
        mesh=mesh,
        compiler_params=_SC_PARAMS,
        out_type=jax.ShapeDtypeStruct((NI, W), jnp.float32),
        scratch_types=[
            pltpu.VMEM((CH, W), jnp.float32),
            pltpu.VMEM((CH,), jnp.float32),
        ],
    )
    def fuser(iemb_h, ibias_h, fused_h, fuse_v, bias_v):
        w = lax.axis_index("s") * NC + lax.axis_index("c")
        dcol = jnp.full((16,), D, jnp.int32)
        lid = lax.iota(jnp.int32, 16)

        def chunk(i, carry):
            c = w + i * NW

            @pl.when(c < NCH)
            def _():
                sl = pl.ds(c * CH, CH)
                pltpu.sync_copy(iemb_h.at[sl], fuse_v.at[:, pl.ds(0, D)])
                pltpu.sync_copy(ibias_h.at[sl], bias_v)

                def put(g, c2):
                    rows = g * LANES + lid
                    vals = plsc.load_gather(bias_v, [rows])
                    plsc.store_scatter(fuse_v, [rows, dcol], vals)
                    return c2

                lax.fori_loop(0, CH // LANES, put, 0, unroll=False)
                pltpu.sync_copy(fuse_v, fused_h.at[sl])

            return carry

        lax.fori_loop(0, ITER, chunk, 0, unroll=False)

    return fuser


def _build_kernel(B, L, D, S):
    NW = NC * NS
    UPW = B // NW
    PPW = UPW * L
    NSC = PPW // S
    KI = S // IDX_CHUNK
    NG = S // LANES
    assert NSC % 2 == 0 and NSC >= 4

    mesh = plsc.VectorSubcoreMesh(core_axis_name="c", subcore_axis_name="s")

    @functools.partial(
        pl.kernel,
        mesh=mesh,
        compiler_params=_SC_PARAMS,
        out_type=jax.ShapeDtypeStruct((B * L,), jnp.float32),
        scratch_types=[
            pltpu.VMEM((UPW, D), jnp.float32),
            pltpu.VMEM((UPW,), jnp.float32),
            pltpu.VMEM((2, S), jnp.int32),
            pltpu.VMEM((2, S, W), jnp.float32),
            pltpu.VMEM((2, S), jnp.float32),
            pltpu.SemaphoreType.DMA,
            pltpu.SemaphoreType.DMA,
            pltpu.SemaphoreType.DMA,
            pltpu.SemaphoreType.DMA,
        ],
    )
    def mf_kernel(items_h, ifused_h, ueb_h, ubb_h, out_h,
                  ue_v, ub_v, idx_v, irows_v, out_v,
                  gsem0, gsem1, osem0, osem1):
        w = lax.axis_index("s") * NC + lax.axis_index("c")
        ubase = w * UPW
        pbase0 = w * PPW
        gsem = (gsem0, gsem1)
        osem = (osem0, osem1)

        def fire_gathers(sc, b):
            pbase = pbase0 + sc * S
            pltpu.sync_copy(items_h.at[pl.ds(pbase, S)], idx_v.at[b])
            for k in range(KI):
                sl = pl.ds(k * IDX_CHUNK, IDX_CHUNK)
                idx = idx_v.at[b, sl]
                pltpu.async_copy(ifused_h.at[idx], irows_v.at[b, sl], gsem[b])

        def drain_gathers(sc, b):
            for k in range(KI):
                sl = pl.ds(k * IDX_CHUNK, IDX_CHUNK)
                idx = idx_v.at[b, sl]
                pltpu.make_async_copy(
                    ifused_h.at[idx], irows_v.at[b, sl], gsem[b]).wait()

        pltpu.sync_copy(ueb_h.at[pl.ds(ubase, UPW)], ue_v)
        pltpu.sync_copy(ubb_h.at[pl.ds(ubase, UPW)], ub_v)
        fire_gathers(jnp.int32(0), 0)
        fire_gathers(jnp.int32(1), 1)

        lid = lax.iota(jnp.int32, 16)

        def body(sc, b):
            drain_gathers(sc, b)

            @pl.when(sc >= 2)
            def _():
                pltpu.make_async_copy(
                    out_v.at[b],
                    out_h.at[pl.ds(pbase0 + (sc - 2) * S, S)],
                    osem[b]).wait()

            def group(g, c2):
                p_local = g * LANES + lid
                p_worker = sc * S + p_local
                u_loc = lax.div(p_worker, jnp.int32(L))
                acc = plsc.load_gather(ub_v, [u_loc])
                acc = acc + plsc.load_gather(
                    irows_v.at[b], [p_local, jnp.full((16,), D, jnp.int32)])
                for d in range(D):
                    dv = jnp.full((16,), d, jnp.int32)
                    ie = plsc.load_gather(irows_v.at[b], [p_local, dv])
                    ue = plsc.load_gather(ue_v, [u_loc, dv])
                    acc = acc + ie * ue
                out_v[b, pl.ds(g * LANES, LANES)] = acc
                return c2

            lax.fori_loop(0, NG, group, 0, unroll=False)
            pltpu.async_copy(
                out_v.at[b], out_h.at[pl.ds(pbase0 + sc * S, S)], osem[b])

            @pl.when(sc + 2 < NSC)
            def _():
                fire_gathers(sc + 2, b)

        def pair_body(sc2, carry):
            body(2 * sc2, 0)
            body(2 * sc2 + 1, 1)
            return carry

        lax.fori_loop(0, NSC // 2, pair_body, 0, unroll=False)

        for b in range(2):
            pltpu.make_async_copy(
                out_v.at[b],
                out_h.at[pl.ds(pbase0 + (NSC - 2 + b) * S, S)],
                osem[b]).wait()

    return mf_kernel


def kernel(users, items, user_emb, item_emb, user_bias, item_bias, bias):
    B, L = items.shape
    NI, D = item_emb.shape
    users = users.astype(jnp.int32)
    items_flat = items.astype(jnp.int32).reshape(-1)
    ue_b = jnp.take(user_emb, users, axis=0)
    ub_b = jnp.take(user_bias, users, axis=0) + bias[0]
    ifused = _build_fuser(NI, D)(item_emb, item_bias)
    fn = _build_kernel(B, L, D, S=640)
    out_flat = fn(items_flat, ifused, ue_b, ub_b)
    return out_flat.reshape(B, L)


# speedup vs baseline: 1.2014x; 1.2014x over previous
import functools

import jax
import jax.numpy as jnp
from jax import lax
from jax.experimental import pallas as pl
from jax.experimental.pallas import tpu as pltpu
from jax.experimental.pallas import tpu_sc as plsc

NC = 2
NS = 16
LANES = 16
IDX_CHUNK = 128
W = 40
CH = 800

_SC_PARAMS = pltpu.CompilerParams(
    needs_layout_passes=False, use_tc_tiling_on_sc=False)


def _build_fuser(NI, D):
    NW = NC * NS
    NCH = NI // CH
    ITER = -(-NCH // NW)
    mesh = plsc.VectorSubcoreMesh(core_axis_name="c", subcore_axis_name="s")

    @functools.partial(
        pl.kernel,
        mesh=mesh,
        compiler_params=_SC_PARAMS,
        out_type=jax.ShapeDtypeStruct((NI, W), jnp.float32),
        scratch_types=[
            pltpu.VMEM((2, CH, W), jnp.float32),
            pltpu.VMEM((2, CH), jnp.float32),
            pltpu.SemaphoreType.DMA,
            pltpu.SemaphoreType.DMA,
            pltpu.SemaphoreType.DMA,
            pltpu.SemaphoreType.DMA,
        ],
    )
    def fuser(iemb_h, ibias_h, fused_h, fuse_v, bias_v,
              isem0, isem1, osem0, osem1):
        w = lax.axis_index("s") * NC + lax.axis_index("c")
        dcol = jnp.full((16,), D, jnp.int32)
        lid = lax.iota(jnp.int32, 16)
        isem = (isem0, isem1)
        osem = (osem0, osem1)
        IT2 = -(-ITER // 2)

        def fire_in(i, b):
            c = w + i * NW

            @pl.when(c < NCH)
            def _():
                sl = pl.ds(c * CH, CH)
                pltpu.async_copy(
                    iemb_h.at[sl], fuse_v.at[b, :, pl.ds(0, D)], isem[b])
                pltpu.async_copy(ibias_h.at[sl], bias_v.at[b], isem[b])

        def body(i2, i, b):
            c = w + i * NW
            cp = w + (i - 2) * NW

            @pl.when(c < NCH)
            def _():
                sl = pl.ds(c * CH, CH)
                pltpu.make_async_copy(
                    iemb_h.at[sl], fuse_v.at[b, :, pl.ds(0, D)],
                    isem[b]).wait()
                pltpu.make_async_copy(
                    ibias_h.at[sl], bias_v.at[b], isem[b]).wait()

            @pl.when((i2 >= 1) & (cp < NCH))
            def _():
                pltpu.make_async_copy(
                    fuse_v.at[b], fused_h.at[pl.ds(cp * CH, CH)],
                    osem[b]).wait()

            @pl.when(c < NCH)
            def _():
                def put(g, c2):
                    rows = g * LANES + lid
                    vals = plsc.load_gather(bias_v.at[b], [rows])
                    plsc.store_scatter(fuse_v.at[b], [rows, dcol], vals)
                    return c2

                lax.fori_loop(0, CH // LANES, put, 0, unroll=False)
                pltpu.async_copy(
                    fuse_v.at[b], fused_h.at[pl.ds(c * CH, CH)], osem[b])

            fire_in(i + 2, b)

        fire_in(jnp.int32(0), 0)
        fire_in(jnp.int32(1), 1)

        def pair(i2, carry):
            body(i2, 2 * i2, 0)
            body(i2, 2 * i2 + 1, 1)
            return carry

        lax.fori_loop(0, IT2, pair, 0, unroll=False)

        for b in range(2):
            cl = w + (2 * (IT2 - 1) + b) * NW

            @pl.when(cl < NCH)
            def _():
                pltpu.make_async_copy(
                    fuse_v.at[b], fused_h.at[pl.ds(cl * CH, CH)],
                    osem[b]).wait()

    return fuser


def _build_kernel(B, L, D, S):
    NW = NC * NS
    UPW = B // NW
    PPW = UPW * L
    NSC = PPW // S
    KI = S // IDX_CHUNK
    NG = S // LANES
    assert NSC % 2 == 0 and NSC >= 4

    mesh = plsc.VectorSubcoreMesh(core_axis_name="c", subcore_axis_name="s")

    @functools.partial(
        pl.kernel,
        mesh=mesh,
        compiler_params=_SC_PARAMS,
        out_type=jax.ShapeDtypeStruct((B * L,), jnp.float32),
        scratch_types=[
            pltpu.VMEM((UPW, D), jnp.float32),
            pltpu.VMEM((UPW,), jnp.float32),
            pltpu.VMEM((2, S), jnp.int32),
            pltpu.VMEM((2, S, W), jnp.float32),
            pltpu.VMEM((2, S), jnp.float32),
            pltpu.SemaphoreType.DMA,
            pltpu.SemaphoreType.DMA,
            pltpu.SemaphoreType.DMA,
            pltpu.SemaphoreType.DMA,
        ],
    )
    def mf_kernel(items_h, ifused_h, ueb_h, ubb_h, out_h,
                  ue_v, ub_v, idx_v, irows_v, out_v,
                  gsem0, gsem1, osem0, osem1):
        w = lax.axis_index("s") * NC + lax.axis_index("c")
        ubase = w * UPW
        pbase0 = w * PPW
        gsem = (gsem0, gsem1)
        osem = (osem0, osem1)

        def fire_gathers(sc, b):
            pbase = pbase0 + sc * S
            pltpu.sync_copy(items_h.at[pl.ds(pbase, S)], idx_v.at[b])
            for k in range(KI):
                sl = pl.ds(k * IDX_CHUNK, IDX_CHUNK)
                idx = idx_v.at[b, sl]
                pltpu.async_copy(ifused_h.at[idx], irows_v.at[b, sl], gsem[b])

        def drain_gathers(sc, b):
            for k in range(KI):
                sl = pl.ds(k * IDX_CHUNK, IDX_CHUNK)
                idx = idx_v.at[b, sl]
                pltpu.make_async_copy(
                    ifused_h.at[idx], irows_v.at[b, sl], gsem[b]).wait()

        pltpu.sync_copy(ueb_h.at[pl.ds(ubase, UPW)], ue_v)
        pltpu.sync_copy(ubb_h.at[pl.ds(ubase, UPW)], ub_v)
        fire_gathers(jnp.int32(0), 0)
        fire_gathers(jnp.int32(1), 1)

        lid = lax.iota(jnp.int32, 16)

        def body(sc, b):
            drain_gathers(sc, b)

            @pl.when(sc >= 2)
            def _():
                pltpu.make_async_copy(
                    out_v.at[b],
                    out_h.at[pl.ds(pbase0 + (sc - 2) * S, S)],
                    osem[b]).wait()

            def group(g, c2):
                p_local = g * LANES + lid
                p_worker = sc * S + p_local
                u_loc = lax.div(p_worker, jnp.int32(L))
                acc = plsc.load_gather(ub_v, [u_loc])
                acc = acc + plsc.load_gather(
                    irows_v.at[b], [p_local, jnp.full((16,), D, jnp.int32)])
                for d in range(D):
                    dv = jnp.full((16,), d, jnp.int32)
                    ie = plsc.load_gather(irows_v.at[b], [p_local, dv])
                    ue = plsc.load_gather(ue_v, [u_loc, dv])
                    acc = acc + ie * ue
                out_v[b, pl.ds(g * LANES, LANES)] = acc
                return c2

            lax.fori_loop(0, NG, group, 0, unroll=False)
            pltpu.async_copy(
                out_v.at[b], out_h.at[pl.ds(pbase0 + sc * S, S)], osem[b])

            @pl.when(sc + 2 < NSC)
            def _():
                fire_gathers(sc + 2, b)

        def pair_body(sc2, carry):
            body(2 * sc2, 0)
            body(2 * sc2 + 1, 1)
            return carry

        lax.fori_loop(0, NSC // 2, pair_body, 0, unroll=False)

        for b in range(2):
            pltpu.make_async_copy(
                out_v.at[b],
                out_h.at[pl.ds(pbase0 + (NSC - 2 + b) * S, S)],
                osem[b]).wait()

    return mf_kernel


def kernel(users, items, user_emb, item_emb, user_bias, item_bias, bias):
    B, L = items.shape
    NI, D = item_emb.shape
    users = users.astype(jnp.int32)
    items_flat = items.astype(jnp.int32).reshape(-1)
    ue_b = jnp.take(user_emb, users, axis=0)
    ub_b = jnp.take(user_bias, users, axis=0) + bias[0]
    ifused = _build_fuser(NI, D)(item_emb, item_bias)
    fn = _build_kernel(B, L, D, S=640)
    out_flat = fn(items_flat, ifused, ue_b, ub_b)
    return out_flat.reshape(B, L)
